# Initial kernel scaffold; baseline (speedup 1.0000x reference)
import jax
import jax.numpy as jnp
from jax.experimental import pallas as pl


def kernel(x_author, x_paper, ei_writes, ei_rev_writes, Win_author, bin_author, Wout_author, bout_author, Win_paper, bin_paper, Wout_paper, bout_paper, Wk0_author, Wq0_author, Wv0_author, Wa0_author, skip0_author, Wk0_paper, Wq0_paper, Wv0_paper, Wa0_paper, skip0_paper, arel0_writes, mrel0_writes, prel0_writes, arel0_rev_writes, mrel0_rev_writes, prel0_rev_writes, Wk1_author, Wq1_author, Wv1_author, Wa1_author, skip1_author, Wk1_paper, Wq1_paper, Wv1_paper, Wa1_paper, skip1_paper, arel1_writes, mrel1_writes, prel1_writes, arel1_rev_writes, mrel1_rev_writes, prel1_rev_writes):
    raise NotImplementedError("write your pallas kernel here")


# speedup vs baseline: 3.4205x; 3.4205x over previous
import functools

import jax
import jax.numpy as jnp
import numpy as np
from jax import lax
from jax.experimental import pallas as pl
from jax.experimental.pallas import tpu as pltpu
from jax.experimental.pallas import tpu_sc as plsc

N = 50000
NPAD = 50176
E = 250000
EP = 250880
HEADS, D, HID = 4, 32, 128
EPT = EP // 32
CA = 112
EPB = EP // 16
CB = 112
NB = 512

_f32 = jnp.float32

_scmesh = plsc.VectorSubcoreMesh(core_axis_name="c", subcore_axis_name="s",
                                 num_cores=2, num_subcores=16)


def _in_proj_body(x_ref, w_ref, b_ref, o_ref):
    i = pl.program_id(0)
    y = jnp.dot(x_ref[...], w_ref[...], preferred_element_type=_f32) + b_ref[...]
    y = jnp.maximum(y, 0.0)
    rows = i * NB + lax.broadcasted_iota(jnp.int32, (NB, HID), 0)
    o_ref[...] = jnp.where(rows < N, y, 0.0)


def _in_proj(x, w, b):
    return pl.pallas_call(
        _in_proj_body,
        grid=(NPAD // NB,),
        in_specs=[pl.BlockSpec((NB, HID), lambda i: (i, 0)),
                  pl.BlockSpec((HID, HID), lambda i: (0, 0)),
                  pl.BlockSpec((1, HID), lambda i: (0, 0))],
        out_specs=pl.BlockSpec((NB, HID), lambda i: (i, 0)),
        out_shape=jax.ShapeDtypeStruct((NPAD, HID), _f32),
    )(x, w, b)


def _qkv_body(x_ref, w_ref, q_ref, k_ref, v_ref):
    y = jnp.dot(x_ref[...], w_ref[...], preferred_element_type=_f32)
    q_ref[...] = y[:, 0:HID]
    k_ref[...] = y[:, HID:2 * HID]
    v_ref[...] = y[:, 2 * HID:3 * HID]


def _qkv(x, wcat):
    return pl.pallas_call(
        _qkv_body,
        grid=(NPAD // NB,),
        in_specs=[pl.BlockSpec((NB, HID), lambda i: (i, 0)),
                  pl.BlockSpec((HID, 3 * HID), lambda i: (0, 0))],
        out_specs=[pl.BlockSpec((NB, HID), lambda i: (i, 0)),
                   pl.BlockSpec((NB, HID), lambda i: (i, 0)),
                   pl.BlockSpec((NB, HID), lambda i: (i, 0))],
        out_shape=[jax.ShapeDtypeStruct((NPAD, HID), _f32),
                   jax.ShapeDtypeStruct((NPAD, HID), _f32),
                   jax.ShapeDtypeStruct((NPAD, HID), _f32)],
    )(x, wcat)


def _agg_body(n_ref, s_ref, x_ref, w_ref, a_ref, o_ref):
    i = pl.program_id(0)
    num = jnp.concatenate([n_ref[h] for h in range(HEADS)], axis=1)
    den = jnp.concatenate(
        [jnp.broadcast_to(s_ref[h][:, None], (NB, D)) for h in range(HEADS)], axis=1)
    agg = num / (den + 1e-16)
    g = jax.nn.gelu(agg)
    out = jnp.dot(g, w_ref[...], preferred_element_type=_f32)
    a = a_ref[0, 0]
    y = a * out + (1.0 - a) * x_ref[...]
    rows = i * NB + lax.broadcasted_iota(jnp.int32, (NB, HID), 0)
    o_ref[...] = jnp.where(rows < N, y, 0.0)


def _agg_update(numer, ssum, x, wa, a_gate):
    return pl.pallas_call(
        _agg_body,
        grid=(NPAD // NB,),
        in_specs=[pl.BlockSpec((HEADS, NB, D), lambda i: (0, i, 0)),
                  pl.BlockSpec((HEADS, NB), lambda i: (0, i)),
                  pl.BlockSpec((NB, HID), lambda i: (i, 0)),
                  pl.BlockSpec((HID, HID), lambda i: (0, 0)),
                  pl.BlockSpec((1, 1), lambda i: (0, 0))],
        out_specs=pl.BlockSpec((NB, HID), lambda i: (i, 0)),
        out_shape=jax.ShapeDtypeStruct((NPAD, HID), _f32),
    )(numer, ssum, x, wa, a_gate)


def _out_proj_body(x_ref, w_ref, b_ref, o_ref):
    y = jnp.dot(x_ref[...], w_ref[...], preferred_element_type=_f32) + b_ref[...]
    o_ref[...] = jnp.maximum(y, 0.0)


def _out_proj(x, w, b):
    return pl.pallas_call(
        _out_proj_body,
        grid=(NPAD // NB,),
        in_specs=[pl.BlockSpec((NB, HID), lambda i: (i, 0)),
                  pl.BlockSpec((HID, HID), lambda i: (0, 0)),
                  pl.BlockSpec((1, HID), lambda i: (0, 0))],
        out_specs=pl.BlockSpec((NB, HID), lambda i: (i, 0)),
        out_shape=jax.ShapeDtypeStruct((N, HID), _f32),
    )(x, w, b)


@functools.partial(
    pl.kernel,
    out_type=jax.ShapeDtypeStruct((HEADS * EP,), _f32),
    mesh=_scmesh,
    scratch_types=[pltpu.VMEM((CA,), jnp.int32),
                   pltpu.VMEM((CA,), jnp.int32),
                   pltpu.VMEM((CA, HID), _f32),
                   pltpu.VMEM((CA, HID), _f32),
                   [pltpu.VMEM((EPT,), _f32) for _ in range(HEADS)],
                   pltpu.SemaphoreType.DMA,
                   pltpu.SemaphoreType.DMA],
    compiler_params=pltpu.CompilerParams(needs_layout_passes=False),
)
def _pass_a(q_hbm, k_hbm, src_hbm, dst_hbm, alpha_out,
            srcv, dstv, krows, qrows, astage, sem1, sem2):
    wid = lax.axis_index("s") * 2 + lax.axis_index("c")
    ebase = wid * EPT
    iota = lax.iota(jnp.int32, 16)

    def chunk_body(g, _):
        cb = ebase + g * CA
        pltpu.sync_copy(src_hbm.at[pl.ds(cb, CA)], srcv)
        pltpu.sync_copy(dst_hbm.at[pl.ds(cb, CA)], dstv)
        cp_k = pltpu.async_copy(k_hbm.at[srcv], krows, sem1)
        cp_q = pltpu.async_copy(q_hbm.at[dstv], qrows, sem2)
        cp_k.wait()
        cp_q.wait()

        def group_body(t, _):
            rows = iota + t * 16
            off = g * CA + t * 16
            for h in range(HEADS):
                acc = None
                for d in range(D):
                    cols = jnp.full((16,), h * D + d, jnp.int32)
                    qv = plsc.load_gather(qrows, [rows, cols])
                    kv = plsc.load_gather(krows, [rows, cols])
                    p = qv * kv
                    acc = p if acc is None else acc + p
                astage[h][pl.ds(off, 16)] = acc
            return 0

        lax.fori_loop(0, CA // 16, group_body, 0)
        return 0

    lax.fori_loop(0, EPT // CA, chunk_body, 0)
    for h in range(HEADS):
        pltpu.sync_copy(astage[h], alpha_out.at[pl.ds(h * EP + ebase, EPT)])


QN = NPAD // 4
JR = 256


@functools.partial(
    pl.kernel,
    out_type=[jax.ShapeDtypeStruct((HEADS * NPAD, D), _f32),
              jax.ShapeDtypeStruct((HEADS * NPAD,), _f32)],
    mesh=_scmesh,
    scratch_types=[pltpu.VMEM((CB,), jnp.int32),
                   pltpu.VMEM((CB,), jnp.int32),
                   pltpu.VMEM((CB,), _f32),
                   pltpu.VMEM((CB, HID), _f32),
                   pltpu.VMEM((CB, D), _f32),
                   pltpu.VMEM((400, D), _f32),
                   pltpu.VMEM((400,), _f32),
                   pltpu.VMEM((400,), jnp.int32),
                   pltpu.VMEM_SHARED((QN + JR, D), _f32),
                   pltpu.VMEM_SHARED((QN + JR,), _f32),
                   pltpu.SemaphoreType.DMA],
    compiler_params=pltpu.CompilerParams(needs_layout_passes=False),
)
def _pass_b(v_hbm, src_hbm, dst_hbm, alpha_hbm, z2_hbm, z1_hbm,
            numer_out, ssum_out, srcv, dstv, ev, vrows, msgbuf, stg, stg1,
            idbuf, acc, accs, sem):
    c = lax.axis_index("c")
    s = lax.axis_index("s")
    iota = lax.iota(jnp.int32, 16)
    zrpt = (QN + JR) // 16

    for p in range(8):
        h = c * 2 + p // 4
        lo = (p % 4) * QN
        pltpu.sync_copy(z2_hbm.at[pl.ds(0, 400)], stg)
        pltpu.sync_copy(z1_hbm.at[pl.ds(0, 400)], stg1)
        for zb in range(2):
            zbase = s * zrpt + zb * 400

            def idfill(i, _):
                idbuf[pl.ds(i * 16, 16)] = iota + (zbase + i * 16)
                return 0

            lax.fori_loop(0, 25, idfill, 0)
            pltpu.sync_copy(stg, acc.at[idbuf])
            pltpu.sync_copy(stg1, accs.at[idbuf])
        plsc.subcore_barrier()

        def chunk_body(g, _):
            base = s * EPB + g * CB
            pltpu.sync_copy(src_hbm.at[pl.ds(base, CB)], srcv)
            pltpu.sync_copy(dst_hbm.at[pl.ds(base, CB)], dstv)
            pltpu.async_copy(v_hbm.at[srcv], vrows, sem).wait()
            pltpu.sync_copy(alpha_hbm.at[pl.ds(h * EP + base, CB)], ev)

            def group_body(t, _):
                rows = iota + t * 16
                dl = dstv[pl.ds(t * 16, 16)] - lo
                msk = (dl >= 0) & (dl < QN)
                junk = QN + ((iota + t * 16) & (JR - 1))
                dstv[pl.ds(t * 16, 16)] = jnp.where(msk, dl, junk)
                e16 = jnp.exp(ev[pl.ds(t * 16, 16)])
                ev[pl.ds(t * 16, 16)] = e16
                for d in range(D):
                    colv = jnp.full((16,), d, jnp.int32) + h * D
                    cold = jnp.full((16,), d, jnp.int32)
                    vv = plsc.load_gather(vrows, [rows, colv])
                    plsc.store_scatter(msgbuf, [rows, cold], vv * e16)
                return 0

            lax.fori_loop(0, CB // 16, group_body, 0)
            pltpu.sync_copy(msgbuf, acc.at[dstv], add=True)
            pltpu.sync_copy(ev, accs.at[dstv], add=True)
            return 0

        lax.fori_loop(0, EPB // CB, chunk_body, 0)
        plsc.subcore_barrier()
        for wb, wn in ((0, 400), (400, 384)):
            wbase = s * (QN // 16) + wb
            slo = pl.ds(h * NPAD + lo + wbase, wn)

            def idfill2(i, _):
                idbuf[pl.ds(i * 16, 16)] = iota + (wbase + i * 16)
                return 0

            lax.fori_loop(0, wn // 16, idfill2, 0)
            pltpu.async_copy(acc.at[idbuf.at[pl.ds(0, wn)]], stg.at[pl.ds(0, wn)], sem).wait()
            pltpu.sync_copy(stg.at[pl.ds(0, wn)], numer_out.at[slo])
            pltpu.async_copy(accs.at[idbuf.at[pl.ds(0, wn)]], stg1.at[pl.ds(0, wn)], sem).wait()
            pltpu.sync_copy(stg1.at[pl.ds(0, wn)], ssum_out.at[slo])
        plsc.subcore_barrier()


def _fold_kv(w, rel, prel=None):
    wh = w.reshape(HID, HEADS, D)
    out = jnp.einsum("ihd,hde->ihe", wh, rel)
    if prel is not None:
        out = out * (prel[None, :, None] / np.sqrt(D))
    return out.reshape(HID, HID)


def _pad_edges(ei):
    src, dst = ei[0], ei[1]
    npad = EP - E
    i = jnp.arange(npad)
    src_p = jnp.concatenate([src, (i % N).astype(src.dtype)])
    dst_p = jnp.concatenate([dst, (N + i % (NPAD - N)).astype(dst.dtype)])
    return src_p.astype(jnp.int32), dst_p.astype(jnp.int32)


def kernel(x_author, x_paper, ei_writes, ei_rev_writes, Win_author, bin_author, Wout_author, bout_author, Win_paper, bin_paper, Wout_paper, bout_paper, Wk0_author, Wq0_author, Wv0_author, Wa0_author, skip0_author, Wk0_paper, Wq0_paper, Wv0_paper, Wa0_paper, skip0_paper, arel0_writes, mrel0_writes, prel0_writes, arel0_rev_writes, mrel0_rev_writes, prel0_rev_writes, Wk1_author, Wq1_author, Wv1_author, Wa1_author, skip1_author, Wk1_paper, Wq1_paper, Wv1_paper, Wa1_paper, skip1_paper, arel1_writes, mrel1_writes, prel1_writes, arel1_rev_writes, mrel1_rev_writes, prel1_rev_writes):
    p = dict(locals())
    types = ["author", "paper"]
    src_rel = {"author": "writes", "paper": "rev_writes"}
    edges = {"writes": _pad_edges(ei_writes), "rev_writes": _pad_edges(ei_rev_writes)}
    z2 = jax.lax.optimization_barrier(jnp.zeros((NPAD, D), _f32))
    z1 = jax.lax.optimization_barrier(jnp.zeros((NPAD,), _f32))

    x = {}
    for t in types:
        xp = jnp.pad(p["x_" + t], ((0, NPAD - N), (0, 0)))
        x[t] = _in_proj(xp, p["Win_" + t], p["bin_" + t].reshape(1, HID))

    for l in range(2):
        q, k, v = {}, {}, {}
        for t in types:
            r = src_rel[t]
            wk = _fold_kv(p["Wk%d_%s" % (l, t)], p["arel%d_%s" % (l, r)],
                          p["prel%d_%s" % (l, r)])
            wv = _fold_kv(p["Wv%d_%s" % (l, t)], p["mrel%d_%s" % (l, r)])
            wcat = jnp.concatenate([p["Wq%d_%s" % (l, t)], wk, wv], axis=1)
            q[t], k[t], v[t] = _qkv(x[t], wcat)

        agg = {}
        for (st, r, dt) in [("author", "writes", "paper"),
                            ("paper", "rev_writes", "author")]:
            src_p, dst_p = edges[r]
            alpha = _pass_a(q[dt], k[st], src_p, dst_p)
            numer, ssum = _pass_b(v[st], src_p, dst_p, alpha, z2, z1)
            agg[dt] = (numer.reshape(HEADS, NPAD, D), ssum.reshape(HEADS, NPAD))

        new_x = {}
        for t in types:
            a_gate = jax.nn.sigmoid(p["skip%d_%s" % (l, t)]).reshape(1, 1)
            numer, ssum = agg[t]
            new_x[t] = _agg_update(numer, ssum, x[t], p["Wa%d_%s" % (l, t)], a_gate)
        x = new_x

    outs = []
    for t in types:
        outs.append(_out_proj(x[t], p["Wout_" + t], p["bout_" + t].reshape(1, HID)))
    return (outs[0], outs[1])


# speedup vs baseline: 4.0317x; 1.1787x over previous
import functools

import jax
import jax.numpy as jnp
import numpy as np
from jax import lax
from jax.experimental import pallas as pl
from jax.experimental.pallas import tpu as pltpu
from jax.experimental.pallas import tpu_sc as plsc

N = 50000
NPAD = 50176
E = 250000
EP = 250880
HEADS, D, HID = 4, 32, 128
EPT = EP // 32
CA = 112
EPB = EP // 16
CB = 112
NB = 512

_f32 = jnp.float32

_scmesh = plsc.VectorSubcoreMesh(core_axis_name="c", subcore_axis_name="s",
                                 num_cores=2, num_subcores=16)


def _in_proj_body(x_ref, w_ref, b_ref, o_ref):
    i = pl.program_id(0)
    y = jnp.dot(x_ref[...], w_ref[...], preferred_element_type=_f32) + b_ref[...]
    y = jnp.maximum(y, 0.0)
    rows = i * NB + lax.broadcasted_iota(jnp.int32, (NB, HID), 0)
    o_ref[...] = jnp.where(rows < N, y, 0.0)


def _in_proj(x, w, b):
    return pl.pallas_call(
        _in_proj_body,
        grid=(NPAD // NB,),
        in_specs=[pl.BlockSpec((NB, HID), lambda i: (i, 0)),
                  pl.BlockSpec((HID, HID), lambda i: (0, 0)),
                  pl.BlockSpec((1, HID), lambda i: (0, 0))],
        out_specs=pl.BlockSpec((NB, HID), lambda i: (i, 0)),
        out_shape=jax.ShapeDtypeStruct((NPAD, HID), _f32),
    )(x, w, b)


def _qkv_body(x_ref, w_ref, q_ref, k_ref, v_ref):
    y = jnp.dot(x_ref[...], w_ref[...], preferred_element_type=_f32)
    q_ref[...] = y[:, 0:HID]
    k_ref[...] = y[:, HID:2 * HID]
    v_ref[...] = y[:, 2 * HID:3 * HID]


def _qkv(x, wcat):
    return pl.pallas_call(
        _qkv_body,
        grid=(NPAD // NB,),
        in_specs=[pl.BlockSpec((NB, HID), lambda i: (i, 0)),
                  pl.BlockSpec((HID, 3 * HID), lambda i: (0, 0))],
        out_specs=[pl.BlockSpec((NB, HID), lambda i: (i, 0)),
                   pl.BlockSpec((NB, HID), lambda i: (i, 0)),
                   pl.BlockSpec((NB, HID), lambda i: (i, 0))],
        out_shape=[jax.ShapeDtypeStruct((NPAD, HID), _f32),
                   jax.ShapeDtypeStruct((NPAD, HID), _f32),
                   jax.ShapeDtypeStruct((NPAD, HID), _f32)],
    )(x, wcat)


def _agg_body(n_ref, s_ref, x_ref, w_ref, a_ref, o_ref):
    i = pl.program_id(0)
    num = jnp.concatenate([n_ref[h] for h in range(HEADS)], axis=1)
    den = jnp.concatenate(
        [jnp.broadcast_to(s_ref[h][:, None], (NB, D)) for h in range(HEADS)], axis=1)
    agg = num / (den + 1e-16)
    g = jax.nn.gelu(agg)
    out = jnp.dot(g, w_ref[...], preferred_element_type=_f32)
    a = a_ref[0, 0]
    y = a * out + (1.0 - a) * x_ref[...]
    rows = i * NB + lax.broadcasted_iota(jnp.int32, (NB, HID), 0)
    o_ref[...] = jnp.where(rows < N, y, 0.0)


def _agg_update(numer, ssum, x, wa, a_gate):
    return pl.pallas_call(
        _agg_body,
        grid=(NPAD // NB,),
        in_specs=[pl.BlockSpec((HEADS, NB, D), lambda i: (0, i, 0)),
                  pl.BlockSpec((HEADS, NB), lambda i: (0, i)),
                  pl.BlockSpec((NB, HID), lambda i: (i, 0)),
                  pl.BlockSpec((HID, HID), lambda i: (0, 0)),
                  pl.BlockSpec((1, 1), lambda i: (0, 0))],
        out_specs=pl.BlockSpec((NB, HID), lambda i: (i, 0)),
        out_shape=jax.ShapeDtypeStruct((NPAD, HID), _f32),
    )(numer, ssum, x, wa, a_gate)


def _out_proj_body(x_ref, w_ref, b_ref, o_ref):
    y = jnp.dot(x_ref[...], w_ref[...], preferred_element_type=_f32) + b_ref[...]
    o_ref[...] = jnp.maximum(y, 0.0)


def _out_proj(x, w, b):
    return pl.pallas_call(
        _out_proj_body,
        grid=(NPAD // NB,),
        in_specs=[pl.BlockSpec((NB, HID), lambda i: (i, 0)),
                  pl.BlockSpec((HID, HID), lambda i: (0, 0)),
                  pl.BlockSpec((1, HID), lambda i: (0, 0))],
        out_specs=pl.BlockSpec((NB, HID), lambda i: (i, 0)),
        out_shape=jax.ShapeDtypeStruct((N, HID), _f32),
    )(x, w, b)


@functools.partial(
    pl.kernel,
    out_type=jax.ShapeDtypeStruct((HEADS * EP,), _f32),
    mesh=_scmesh,
    scratch_types=[[pltpu.VMEM((CA,), jnp.int32) for _ in range(2)],
                   [pltpu.VMEM((CA,), jnp.int32) for _ in range(2)],
                   [pltpu.VMEM((CA, HID), _f32) for _ in range(2)],
                   [pltpu.VMEM((CA, HID), _f32) for _ in range(2)],
                   [pltpu.VMEM((EPT,), _f32) for _ in range(HEADS)],
                   [pltpu.SemaphoreType.DMA for _ in range(2)],
                   [pltpu.SemaphoreType.DMA for _ in range(2)]],
    compiler_params=pltpu.CompilerParams(needs_layout_passes=False),
)
def _pass_a(q_hbm, k_hbm, src_hbm, dst_hbm, alpha_out,
            srcv, dstv, krows, qrows, astage, semi, semg):
    wid = lax.axis_index("s") * 2 + lax.axis_index("c")
    ebase = wid * EPT
    iota = lax.iota(jnp.int32, 16)
    NCHUNK = EPT // CA

    pltpu.sync_copy(src_hbm.at[pl.ds(ebase, CA)], srcv[0])
    pltpu.sync_copy(dst_hbm.at[pl.ds(ebase, CA)], dstv[0])
    pltpu.async_copy(k_hbm.at[srcv[0]], krows[0], semg[0])
    pltpu.async_copy(q_hbm.at[dstv[0]], qrows[0], semg[0])

    def do_chunk(g, b):
        nxt = 1 - b
        gn = jnp.minimum(g + 1, NCHUNK - 1)
        cbn = ebase + gn * CA
        pltpu.async_copy(src_hbm.at[pl.ds(cbn, CA)], srcv[nxt], semi[nxt])
        pltpu.async_copy(dst_hbm.at[pl.ds(cbn, CA)], dstv[nxt], semi[nxt])
        pltpu.make_async_copy(k_hbm.at[srcv[b]], krows[b], semg[b]).wait()
        pltpu.make_async_copy(q_hbm.at[dstv[b]], qrows[b], semg[b]).wait()

        def group_body(t, _):
            rows = iota + t * 16
            off = g * CA + t * 16
            for h in range(HEADS):
                acc = None
                for d in range(D):
                    cols = jnp.full((16,), h * D + d, jnp.int32)
                    qv = plsc.load_gather(qrows[b], [rows, cols])
                    kv = plsc.load_gather(krows[b], [rows, cols])
                    p = qv * kv
                    acc = p if acc is None else acc + p
                astage[h][pl.ds(off, 16)] = acc
            return 0

        lax.fori_loop(0, CA // 16, group_body, 0)
        pltpu.make_async_copy(src_hbm.at[pl.ds(0, CA)], srcv[nxt], semi[nxt]).wait()
        pltpu.make_async_copy(dst_hbm.at[pl.ds(0, CA)], dstv[nxt], semi[nxt]).wait()
        pltpu.async_copy(k_hbm.at[srcv[nxt]], krows[nxt], semg[nxt])
        pltpu.async_copy(q_hbm.at[dstv[nxt]], qrows[nxt], semg[nxt])

    def pair_body(j, _):
        do_chunk(2 * j, 0)
        do_chunk(2 * j + 1, 1)
        return 0

    lax.fori_loop(0, NCHUNK // 2, pair_body, 0)
    pltpu.make_async_copy(k_hbm.at[srcv[0]], krows[0], semg[0]).wait()
    pltpu.make_async_copy(q_hbm.at[dstv[0]], qrows[0], semg[0]).wait()
    for h in range(HEADS):
        pltpu.sync_copy(astage[h], alpha_out.at[pl.ds(h * EP + ebase, EPT)])


QN = NPAD // 4
JR = 256


@functools.partial(
    pl.kernel,
    out_type=[jax.ShapeDtypeStruct((HEADS * NPAD, D), _f32),
              jax.ShapeDtypeStruct((HEADS * NPAD,), _f32)],
    mesh=_scmesh,
    scratch_types=[[pltpu.VMEM((CB,), jnp.int32) for _ in range(2)],
                   [pltpu.VMEM((CB,), jnp.int32) for _ in range(2)],
                   [pltpu.VMEM((CB,), _f32) for _ in range(2)],
                   [pltpu.VMEM((CB, HID), _f32) for _ in range(2)],
                   pltpu.VMEM((CB, D), _f32),
                   pltpu.VMEM((400, D), _f32),
                   pltpu.VMEM((400,), _f32),
                   pltpu.VMEM((400,), jnp.int32),
                   pltpu.VMEM_SHARED((QN + JR, D), _f32),
                   pltpu.VMEM_SHARED((QN + JR,), _f32),
                   [pltpu.SemaphoreType.DMA for _ in range(2)],
                   [pltpu.SemaphoreType.DMA for _ in range(2)],
                   pltpu.SemaphoreType.DMA],
    compiler_params=pltpu.CompilerParams(needs_layout_passes=False),
)
def _pass_b(v_hbm, src_hbm, dst_hbm, alpha_hbm, z2_hbm, z1_hbm,
            numer_out, ssum_out, srcv, dstv, ev, vrows, msgbuf, stg, stg1,
            idbuf, acc, accs, semi, semg, sem):
    c = lax.axis_index("c")
    s = lax.axis_index("s")
    iota = lax.iota(jnp.int32, 16)
    zrpt = (QN + JR) // 16
    NCHUNK = EPB // CB

    def issue_idx(g, b):
        base = s * EPB + g * CB
        cp1 = pltpu.async_copy(src_hbm.at[pl.ds(base, CB)], srcv[b], semi[b])
        cp2 = pltpu.async_copy(dst_hbm.at[pl.ds(base, CB)], dstv[b], semi[b])
        return cp1, cp2

    for p in range(8):
        h = c * 2 + p // 4
        lo = (p % 4) * QN
        pltpu.sync_copy(z2_hbm.at[pl.ds(0, 400)], stg)
        pltpu.sync_copy(z1_hbm.at[pl.ds(0, 400)], stg1)
        for zb in range(2):
            zbase = s * zrpt + zb * 400

            def idfill(i, _):
                idbuf[pl.ds(i * 16, 16)] = iota + (zbase + i * 16)
                return 0

            lax.fori_loop(0, 25, idfill, 0)
            pltpu.sync_copy(stg, acc.at[idbuf])
            pltpu.sync_copy(stg1, accs.at[idbuf])
        plsc.subcore_barrier()

        base0 = s * EPB
        pltpu.sync_copy(src_hbm.at[pl.ds(base0, CB)], srcv[0])
        pltpu.sync_copy(dst_hbm.at[pl.ds(base0, CB)], dstv[0])
        pltpu.sync_copy(alpha_hbm.at[pl.ds(h * EP + base0, CB)], ev[0])
        pltpu.async_copy(v_hbm.at[srcv[0]], vrows[0], semg[0])

        def do_chunk(g, b):
            nxt = 1 - b
            gn = jnp.minimum(g + 1, NCHUNK - 1)
            issue_idx(gn, nxt)
            base_n = s * EPB + gn * CB
            pltpu.async_copy(
                alpha_hbm.at[pl.ds(h * EP + base_n, CB)], ev[nxt], semi[nxt])
            pltpu.make_async_copy(v_hbm.at[srcv[b]], vrows[b], semg[b]).wait()

            def group_body(t, _):
                rows = iota + t * 16
                dl = dstv[b][pl.ds(t * 16, 16)] - lo
                msk = (dl >= 0) & (dl < QN)
                junk = QN + ((iota + t * 16) & (JR - 1))
                dstv[b][pl.ds(t * 16, 16)] = jnp.where(msk, dl, junk)
                e16 = jnp.exp(ev[b][pl.ds(t * 16, 16)])
                ev[b][pl.ds(t * 16, 16)] = e16
                for d in range(D):
                    colv = jnp.full((16,), d, jnp.int32) + h * D
                    cold = jnp.full((16,), d, jnp.int32)
                    vv = plsc.load_gather(vrows[b], [rows, colv])
                    plsc.store_scatter(msgbuf, [rows, cold], vv * e16)
                return 0

            lax.fori_loop(0, CB // 16, group_body, 0)
            pltpu.sync_copy(msgbuf, acc.at[dstv[b]], add=True)
            pltpu.sync_copy(ev[b], accs.at[dstv[b]], add=True)
            pltpu.make_async_copy(src_hbm.at[pl.ds(0, CB)], srcv[nxt], semi[nxt]).wait()
            pltpu.make_async_copy(dst_hbm.at[pl.ds(0, CB)], dstv[nxt], semi[nxt]).wait()
            pltpu.make_async_copy(
                alpha_hbm.at[pl.ds(0, CB)], ev[nxt], semi[nxt]).wait()
            pltpu.async_copy(v_hbm.at[srcv[nxt]], vrows[nxt], semg[nxt])

        def pair_body(j, _):
            do_chunk(2 * j, 0)
            do_chunk(2 * j + 1, 1)
            return 0

        lax.fori_loop(0, NCHUNK // 2, pair_body, 0)
        pltpu.make_async_copy(v_hbm.at[srcv[0]], vrows[0], semg[0]).wait()
        plsc.subcore_barrier()
        for wb, wn in ((0, 400), (400, 384)):
            wbase = s * (QN // 16) + wb
            slo = pl.ds(h * NPAD + lo + wbase, wn)

            def idfill2(i, _):
                idbuf[pl.ds(i * 16, 16)] = iota + (wbase + i * 16)
                return 0

            lax.fori_loop(0, wn // 16, idfill2, 0)
            pltpu.async_copy(acc.at[idbuf.at[pl.ds(0, wn)]], stg.at[pl.ds(0, wn)], sem).wait()
            pltpu.sync_copy(stg.at[pl.ds(0, wn)], numer_out.at[slo])
            pltpu.async_copy(accs.at[idbuf.at[pl.ds(0, wn)]], stg1.at[pl.ds(0, wn)], sem).wait()
            pltpu.sync_copy(stg1.at[pl.ds(0, wn)], ssum_out.at[slo])
        plsc.subcore_barrier()


def _fold_kv(w, rel, prel=None):
    wh = w.reshape(HID, HEADS, D)
    out = jnp.einsum("ihd,hde->ihe", wh, rel)
    if prel is not None:
        out = out * (prel[None, :, None] / np.sqrt(D))
    return out.reshape(HID, HID)


def _pad_edges(ei):
    src, dst = ei[0], ei[1]
    npad = EP - E
    i = jnp.arange(npad)
    src_p = jnp.concatenate([src, (i % N).astype(src.dtype)])
    dst_p = jnp.concatenate([dst, (N + i % (NPAD - N)).astype(dst.dtype)])
    return src_p.astype(jnp.int32), dst_p.astype(jnp.int32)


def kernel(x_author, x_paper, ei_writes, ei_rev_writes, Win_author, bin_author, Wout_author, bout_author, Win_paper, bin_paper, Wout_paper, bout_paper, Wk0_author, Wq0_author, Wv0_author, Wa0_author, skip0_author, Wk0_paper, Wq0_paper, Wv0_paper, Wa0_paper, skip0_paper, arel0_writes, mrel0_writes, prel0_writes, arel0_rev_writes, mrel0_rev_writes, prel0_rev_writes, Wk1_author, Wq1_author, Wv1_author, Wa1_author, skip1_author, Wk1_paper, Wq1_paper, Wv1_paper, Wa1_paper, skip1_paper, arel1_writes, mrel1_writes, prel1_writes, arel1_rev_writes, mrel1_rev_writes, prel1_rev_writes):
    p = dict(locals())
    types = ["author", "paper"]
    src_rel = {"author": "writes", "paper": "rev_writes"}
    edges = {"writes": _pad_edges(ei_writes), "rev_writes": _pad_edges(ei_rev_writes)}
    z2 = jax.lax.optimization_barrier(jnp.zeros((NPAD, D), _f32))
    z1 = jax.lax.optimization_barrier(jnp.zeros((NPAD,), _f32))

    x = {}
    for t in types:
        xp = jnp.pad(p["x_" + t], ((0, NPAD - N), (0, 0)))
        x[t] = _in_proj(xp, p["Win_" + t], p["bin_" + t].reshape(1, HID))

    for l in range(2):
        q, k, v = {}, {}, {}
        for t in types:
            r = src_rel[t]
            wk = _fold_kv(p["Wk%d_%s" % (l, t)], p["arel%d_%s" % (l, r)],
                          p["prel%d_%s" % (l, r)])
            wv = _fold_kv(p["Wv%d_%s" % (l, t)], p["mrel%d_%s" % (l, r)])
            wcat = jnp.concatenate([p["Wq%d_%s" % (l, t)], wk, wv], axis=1)
            q[t], k[t], v[t] = _qkv(x[t], wcat)

        agg = {}
        for (st, r, dt) in [("author", "writes", "paper"),
                            ("paper", "rev_writes", "author")]:
            src_p, dst_p = edges[r]
            alpha = _pass_a(q[dt], k[st], src_p, dst_p)
            numer, ssum = _pass_b(v[st], src_p, dst_p, alpha, z2, z1)
            agg[dt] = (numer.reshape(HEADS, NPAD, D), ssum.reshape(HEADS, NPAD))

        new_x = {}
        for t in types:
            a_gate = jax.nn.sigmoid(p["skip%d_%s" % (l, t)]).reshape(1, 1)
            numer, ssum = agg[t]
            new_x[t] = _agg_update(numer, ssum, x[t], p["Wa%d_%s" % (l, t)], a_gate)
        x = new_x

    outs = []
    for t in types:
        outs.append(_out_proj(x[t], p["Wout_" + t], p["bout_" + t].reshape(1, HID)))
    return (outs[0], outs[1])


# speedup vs baseline: 4.1300x; 1.0244x over previous
import functools

import jax
import jax.numpy as jnp
import numpy as np
from jax import lax
from jax.experimental import pallas as pl
from jax.experimental.pallas import tpu as pltpu
from jax.experimental.pallas import tpu_sc as plsc

N = 50000
NPAD = 50176
E = 250000
EP = 250880
HEADS, D, HID = 4, 32, 128
EPT = EP // 32
CA = 112
EPB = EP // 16
CB = 112
NB = 512

_f32 = jnp.float32

_scmesh = plsc.VectorSubcoreMesh(core_axis_name="c", subcore_axis_name="s",
                                 num_cores=2, num_subcores=16)


def _in_proj_body(x_ref, w_ref, b_ref, o_ref):
    i = pl.program_id(0)
    y = jnp.dot(x_ref[...], w_ref[...], preferred_element_type=_f32) + b_ref[...]
    y = jnp.maximum(y, 0.0)
    rows = i * NB + lax.broadcasted_iota(jnp.int32, (NB, HID), 0)
    o_ref[...] = jnp.where(rows < N, y, 0.0)


def _in_proj(x, w, b):
    return pl.pallas_call(
        _in_proj_body,
        grid=(NPAD // NB,),
        in_specs=[pl.BlockSpec((NB, HID), lambda i: (i, 0)),
                  pl.BlockSpec((HID, HID), lambda i: (0, 0)),
                  pl.BlockSpec((1, HID), lambda i: (0, 0))],
        out_specs=pl.BlockSpec((NB, HID), lambda i: (i, 0)),
        out_shape=jax.ShapeDtypeStruct((NPAD, HID), _f32),
    )(x, w, b)


def _qkv_body(x_ref, w_ref, q_ref, k_ref, v_ref):
    y = jnp.dot(x_ref[...], w_ref[...], preferred_element_type=_f32)
    q_ref[...] = y[:, 0:HID]
    k_ref[...] = y[:, HID:2 * HID]
    v_ref[...] = y[:, 2 * HID:3 * HID]


def _qkv(x, wcat):
    return pl.pallas_call(
        _qkv_body,
        grid=(NPAD // NB,),
        in_specs=[pl.BlockSpec((NB, HID), lambda i: (i, 0)),
                  pl.BlockSpec((HID, 3 * HID), lambda i: (0, 0))],
        out_specs=[pl.BlockSpec((NB, HID), lambda i: (i, 0)),
                   pl.BlockSpec((NB, HID), lambda i: (i, 0)),
                   pl.BlockSpec((NB, HID), lambda i: (i, 0))],
        out_shape=[jax.ShapeDtypeStruct((NPAD, HID), _f32),
                   jax.ShapeDtypeStruct((NPAD, HID), _f32),
                   jax.ShapeDtypeStruct((NPAD, HID), _f32)],
    )(x, wcat)


def _agg_body(n_ref, s_ref, x_ref, w_ref, a_ref, o_ref):
    i = pl.program_id(0)
    num = jnp.concatenate([n_ref[h] for h in range(HEADS)], axis=1)
    den = jnp.concatenate(
        [jnp.broadcast_to(s_ref[h][:, None], (NB, D)) for h in range(HEADS)], axis=1)
    agg = num / (den + 1e-16)
    g = jax.nn.gelu(agg)
    out = jnp.dot(g, w_ref[...], preferred_element_type=_f32)
    a = a_ref[0, 0]
    y = a * out + (1.0 - a) * x_ref[...]
    rows = i * NB + lax.broadcasted_iota(jnp.int32, (NB, HID), 0)
    o_ref[...] = jnp.where(rows < N, y, 0.0)


def _agg_update(numer, ssum, x, wa, a_gate):
    return pl.pallas_call(
        _agg_body,
        grid=(NPAD // NB,),
        in_specs=[pl.BlockSpec((HEADS, NB, D), lambda i: (0, i, 0)),
                  pl.BlockSpec((HEADS, NB), lambda i: (0, i)),
                  pl.BlockSpec((NB, HID), lambda i: (i, 0)),
                  pl.BlockSpec((HID, HID), lambda i: (0, 0)),
                  pl.BlockSpec((1, 1), lambda i: (0, 0))],
        out_specs=pl.BlockSpec((NB, HID), lambda i: (i, 0)),
        out_shape=jax.ShapeDtypeStruct((NPAD, HID), _f32),
    )(numer, ssum, x, wa, a_gate)


def _out_proj_body(x_ref, w_ref, b_ref, o_ref):
    y = jnp.dot(x_ref[...], w_ref[...], preferred_element_type=_f32) + b_ref[...]
    o_ref[...] = jnp.maximum(y, 0.0)


def _out_proj(x, w, b):
    return pl.pallas_call(
        _out_proj_body,
        grid=(NPAD // NB,),
        in_specs=[pl.BlockSpec((NB, HID), lambda i: (i, 0)),
                  pl.BlockSpec((HID, HID), lambda i: (0, 0)),
                  pl.BlockSpec((1, HID), lambda i: (0, 0))],
        out_specs=pl.BlockSpec((NB, HID), lambda i: (i, 0)),
        out_shape=jax.ShapeDtypeStruct((N, HID), _f32),
    )(x, w, b)


@functools.partial(
    pl.kernel,
    out_type=jax.ShapeDtypeStruct((HEADS * EP,), _f32),
    mesh=_scmesh,
    scratch_types=[[pltpu.VMEM((CA,), jnp.int32) for _ in range(2)],
                   [pltpu.VMEM((CA,), jnp.int32) for _ in range(2)],
                   [pltpu.VMEM((CA, HID), _f32) for _ in range(2)],
                   [pltpu.VMEM((CA, HID), _f32) for _ in range(2)],
                   [pltpu.VMEM((EPT,), _f32) for _ in range(HEADS)],
                   [pltpu.SemaphoreType.DMA for _ in range(2)],
                   [pltpu.SemaphoreType.DMA for _ in range(2)]],
    compiler_params=pltpu.CompilerParams(needs_layout_passes=False),
)
def _pass_a(q_hbm, k_hbm, src_hbm, dst_hbm, alpha_out,
            srcv, dstv, krows, qrows, astage, semi, semg):
    wid = lax.axis_index("s") * 2 + lax.axis_index("c")
    ebase = wid * EPT
    iota = lax.iota(jnp.int32, 16)
    NCHUNK = EPT // CA

    pltpu.sync_copy(src_hbm.at[pl.ds(ebase, CA)], srcv[0])
    pltpu.sync_copy(dst_hbm.at[pl.ds(ebase, CA)], dstv[0])
    pltpu.async_copy(k_hbm.at[srcv[0]], krows[0], semg[0])
    pltpu.async_copy(q_hbm.at[dstv[0]], qrows[0], semg[0])

    def do_chunk(g, b):
        nxt = 1 - b
        gn = jnp.minimum(g + 1, NCHUNK - 1)
        cbn = ebase + gn * CA
        pltpu.async_copy(src_hbm.at[pl.ds(cbn, CA)], srcv[nxt], semi[nxt])
        pltpu.async_copy(dst_hbm.at[pl.ds(cbn, CA)], dstv[nxt], semi[nxt])
        pltpu.make_async_copy(k_hbm.at[srcv[b]], krows[b], semg[b]).wait()
        pltpu.make_async_copy(q_hbm.at[dstv[b]], qrows[b], semg[b]).wait()

        def group_body(t, _):
            rows = iota + t * 16
            off = g * CA + t * 16
            for h in range(HEADS):
                acc = None
                for d in range(D):
                    cols = jnp.full((16,), h * D + d, jnp.int32)
                    qv = plsc.load_gather(qrows[b], [rows, cols])
                    kv = plsc.load_gather(krows[b], [rows, cols])
                    p = qv * kv
                    acc = p if acc is None else acc + p
                astage[h][pl.ds(off, 16)] = acc
            return 0

        lax.fori_loop(0, CA // 16, group_body, 0)
        pltpu.make_async_copy(src_hbm.at[pl.ds(0, CA)], srcv[nxt], semi[nxt]).wait()
        pltpu.make_async_copy(dst_hbm.at[pl.ds(0, CA)], dstv[nxt], semi[nxt]).wait()
        pltpu.async_copy(k_hbm.at[srcv[nxt]], krows[nxt], semg[nxt])
        pltpu.async_copy(q_hbm.at[dstv[nxt]], qrows[nxt], semg[nxt])

    def pair_body(j, _):
        do_chunk(2 * j, 0)
        do_chunk(2 * j + 1, 1)
        return 0

    lax.fori_loop(0, NCHUNK // 2, pair_body, 0)
    pltpu.make_async_copy(k_hbm.at[srcv[0]], krows[0], semg[0]).wait()
    pltpu.make_async_copy(q_hbm.at[dstv[0]], qrows[0], semg[0]).wait()
    for h in range(HEADS):
        pltpu.sync_copy(astage[h], alpha_out.at[pl.ds(h * EP + ebase, EPT)])


QN = NPAD // 4
JR = 256


@functools.partial(
    pl.kernel,
    out_type=[jax.ShapeDtypeStruct((HEADS * NPAD, D), _f32),
              jax.ShapeDtypeStruct((HEADS * NPAD,), _f32)],
    mesh=_scmesh,
    scratch_types=[[pltpu.VMEM((CB,), jnp.int32) for _ in range(2)],
                   [pltpu.VMEM((CB,), jnp.int32) for _ in range(2)],
                   [pltpu.VMEM((CB,), _f32) for _ in range(2)],
                   [pltpu.VMEM((CB, HID), _f32) for _ in range(2)],
                   [pltpu.VMEM((CB, D), _f32) for _ in range(2)],
                   pltpu.VMEM((160, D), _f32),
                   pltpu.VMEM((160,), _f32),
                   pltpu.VMEM((160,), jnp.int32),
                   pltpu.VMEM_SHARED((QN + JR, D), _f32),
                   pltpu.VMEM_SHARED((QN + JR,), _f32),
                   [pltpu.SemaphoreType.DMA for _ in range(2)],
                   [pltpu.SemaphoreType.DMA for _ in range(2)],
                   [pltpu.SemaphoreType.DMA for _ in range(2)],
                   pltpu.SemaphoreType.DMA],
    compiler_params=pltpu.CompilerParams(needs_layout_passes=False),
)
def _pass_b(v_hbm, src_hbm, dst_hbm, alpha_hbm, z2_hbm, z1_hbm,
            numer_out, ssum_out, srcv, dstv, ev, vrows, msgbuf, stg, stg1,
            idbuf, acc, accs, semi, semg, sema, sem):
    c = lax.axis_index("c")
    s = lax.axis_index("s")
    iota = lax.iota(jnp.int32, 16)
    zrpt = (QN + JR) // 16
    NCHUNK = EPB // CB

    def issue_idx(g, b):
        base = s * EPB + g * CB
        cp1 = pltpu.async_copy(src_hbm.at[pl.ds(base, CB)], srcv[b], semi[b])
        cp2 = pltpu.async_copy(dst_hbm.at[pl.ds(base, CB)], dstv[b], semi[b])
        return cp1, cp2

    def phase_body(p, _):
        h = c * 2 + p // 4
        lo = (p % 4) * QN
        pltpu.sync_copy(z2_hbm.at[pl.ds(0, 160)], stg)
        pltpu.sync_copy(z1_hbm.at[pl.ds(0, 160)], stg1)
        for zb in range(5):
            zbase = s * zrpt + zb * 160

            def idfill(i, _):
                idbuf[pl.ds(i * 16, 16)] = iota + (zbase + i * 16)
                return 0

            lax.fori_loop(0, 10, idfill, 0)
            pltpu.sync_copy(stg, acc.at[idbuf])
            pltpu.sync_copy(stg1, accs.at[idbuf])
        plsc.subcore_barrier()

        base0 = s * EPB
        pltpu.sync_copy(src_hbm.at[pl.ds(base0, CB)], srcv[0])
        pltpu.sync_copy(dst_hbm.at[pl.ds(base0, CB)], dstv[0])
        pltpu.sync_copy(alpha_hbm.at[pl.ds(h * EP + base0, CB)], ev[0])
        pltpu.async_copy(v_hbm.at[srcv[0]], vrows[0], semg[0])

        def do_chunk(g, b, first=False):
            nxt = 1 - b
            gn = jnp.minimum(g + 1, NCHUNK - 1)
            issue_idx(gn, nxt)
            base_n = s * EPB + gn * CB
            pltpu.async_copy(
                alpha_hbm.at[pl.ds(h * EP + base_n, CB)], ev[nxt], semi[nxt])
            pltpu.make_async_copy(v_hbm.at[srcv[b]], vrows[b], semg[b]).wait()
            if not first:
                pltpu.make_async_copy(msgbuf[b], acc.at[dstv[b]], sema[b]).wait()

            def group_body(t, _):
                rows = iota + t * 16
                dl = dstv[b][pl.ds(t * 16, 16)] - lo
                msk = (dl >= 0) & (dl < QN)
                junk = QN + ((iota + t * 16) & (JR - 1))
                dstv[b][pl.ds(t * 16, 16)] = jnp.where(msk, dl, junk)
                e16 = jnp.exp(ev[b][pl.ds(t * 16, 16)])
                ev[b][pl.ds(t * 16, 16)] = e16
                for d in range(D):
                    colv = jnp.full((16,), d, jnp.int32) + h * D
                    cold = jnp.full((16,), d, jnp.int32)
                    vv = plsc.load_gather(vrows[b], [rows, colv])
                    plsc.store_scatter(msgbuf[b], [rows, cold], vv * e16)
                return 0

            lax.fori_loop(0, CB // 16, group_body, 0)
            pltpu.async_copy(msgbuf[b], acc.at[dstv[b]], sema[b], add=True)
            pltpu.sync_copy(ev[b], accs.at[dstv[b]], add=True)
            pltpu.make_async_copy(src_hbm.at[pl.ds(0, CB)], srcv[nxt], semi[nxt]).wait()
            pltpu.make_async_copy(dst_hbm.at[pl.ds(0, CB)], dstv[nxt], semi[nxt]).wait()
            pltpu.make_async_copy(
                alpha_hbm.at[pl.ds(0, CB)], ev[nxt], semi[nxt]).wait()
            pltpu.async_copy(v_hbm.at[srcv[nxt]], vrows[nxt], semg[nxt])

        do_chunk(0, 0, first=True)
        do_chunk(1, 1, first=True)

        def pair_body(j, _):
            do_chunk(2 * j, 0)
            do_chunk(2 * j + 1, 1)
            return 0

        lax.fori_loop(1, NCHUNK // 2, pair_body, 0)
        pltpu.make_async_copy(v_hbm.at[srcv[0]], vrows[0], semg[0]).wait()
        pltpu.make_async_copy(msgbuf[0], acc.at[dstv[0]], sema[0]).wait()
        pltpu.make_async_copy(msgbuf[1], acc.at[dstv[1]], sema[1]).wait()
        plsc.subcore_barrier()
        for wb, wn in ((0, 160), (160, 160), (320, 160), (480, 160), (640, 144)):
            wbase = s * (QN // 16) + wb
            slo = pl.ds(h * NPAD + lo + wbase, wn)

            def idfill2(i, _):
                idbuf[pl.ds(i * 16, 16)] = iota + (wbase + i * 16)
                return 0

            lax.fori_loop(0, wn // 16, idfill2, 0)
            pltpu.async_copy(acc.at[idbuf.at[pl.ds(0, wn)]], stg.at[pl.ds(0, wn)], sem).wait()
            pltpu.sync_copy(stg.at[pl.ds(0, wn)], numer_out.at[slo])
            pltpu.async_copy(accs.at[idbuf.at[pl.ds(0, wn)]], stg1.at[pl.ds(0, wn)], sem).wait()
            pltpu.sync_copy(stg1.at[pl.ds(0, wn)], ssum_out.at[slo])
        plsc.subcore_barrier()
        return 0

    lax.fori_loop(0, 8, phase_body, 0)


def _fold_kv(w, rel, prel=None):
    wh = w.reshape(HID, HEADS, D)
    out = jnp.einsum("ihd,hde->ihe", wh, rel)
    if prel is not None:
        out = out * (prel[None, :, None] / np.sqrt(D))
    return out.reshape(HID, HID)


def _pad_edges(ei):
    src, dst = ei[0], ei[1]
    npad = EP - E
    i = jnp.arange(npad)
    src_p = jnp.concatenate([src, (i % N).astype(src.dtype)])
    dst_p = jnp.concatenate([dst, (N + i % (NPAD - N)).astype(dst.dtype)])
    return src_p.astype(jnp.int32), dst_p.astype(jnp.int32)


def kernel(x_author, x_paper, ei_writes, ei_rev_writes, Win_author, bin_author, Wout_author, bout_author, Win_paper, bin_paper, Wout_paper, bout_paper, Wk0_author, Wq0_author, Wv0_author, Wa0_author, skip0_author, Wk0_paper, Wq0_paper, Wv0_paper, Wa0_paper, skip0_paper, arel0_writes, mrel0_writes, prel0_writes, arel0_rev_writes, mrel0_rev_writes, prel0_rev_writes, Wk1_author, Wq1_author, Wv1_author, Wa1_author, skip1_author, Wk1_paper, Wq1_paper, Wv1_paper, Wa1_paper, skip1_paper, arel1_writes, mrel1_writes, prel1_writes, arel1_rev_writes, mrel1_rev_writes, prel1_rev_writes):
    p = dict(locals())
    types = ["author", "paper"]
    src_rel = {"author": "writes", "paper": "rev_writes"}
    edges = {"writes": _pad_edges(ei_writes), "rev_writes": _pad_edges(ei_rev_writes)}
    z2 = jax.lax.optimization_barrier(jnp.zeros((NPAD, D), _f32))
    z1 = jax.lax.optimization_barrier(jnp.zeros((NPAD,), _f32))

    x = {}
    for t in types:
        xp = jnp.pad(p["x_" + t], ((0, NPAD - N), (0, 0)))
        x[t] = _in_proj(xp, p["Win_" + t], p["bin_" + t].reshape(1, HID))

    for l in range(2):
        q, k, v = {}, {}, {}
        for t in types:
            r = src_rel[t]
            wk = _fold_kv(p["Wk%d_%s" % (l, t)], p["arel%d_%s" % (l, r)],
                          p["prel%d_%s" % (l, r)])
            wv = _fold_kv(p["Wv%d_%s" % (l, t)], p["mrel%d_%s" % (l, r)])
            wcat = jnp.concatenate([p["Wq%d_%s" % (l, t)], wk, wv], axis=1)
            q[t], k[t], v[t] = _qkv(x[t], wcat)

        agg = {}
        for (st, r, dt) in [("author", "writes", "paper"),
                            ("paper", "rev_writes", "author")]:
            src_p, dst_p = edges[r]
            alpha = _pass_a(q[dt], k[st], src_p, dst_p)
            numer, ssum = _pass_b(v[st], src_p, dst_p, alpha, z2, z1)
            agg[dt] = (numer.reshape(HEADS, NPAD, D), ssum.reshape(HEADS, NPAD))

        new_x = {}
        for t in types:
            a_gate = jax.nn.sigmoid(p["skip%d_%s" % (l, t)]).reshape(1, 1)
            numer, ssum = agg[t]
            new_x[t] = _agg_update(numer, ssum, x[t], p["Wa%d_%s" % (l, t)], a_gate)
        x = new_x

    outs = []
    for t in types:
        outs.append(_out_proj(x[t], p["Wout_" + t], p["bout_" + t].reshape(1, HID)))
    return (outs[0], outs[1])


# speedup vs baseline: 5.1050x; 1.2361x over previous
import functools

import jax
import jax.numpy as jnp
import numpy as np
from jax import lax
from jax.experimental import pallas as pl
from jax.experimental.pallas import tpu as pltpu
from jax.experimental.pallas import tpu_sc as plsc

N = 50000
NPAD = 50176
E = 250000
EP = 250880
HEADS, D, HID = 4, 32, 128
EPT = EP // 32
CA = 112
EPB = EP // 16
CB = 112
NB = 512

_f32 = jnp.float32

_scmesh = plsc.VectorSubcoreMesh(core_axis_name="c", subcore_axis_name="s",
                                 num_cores=2, num_subcores=16)


def _in_proj_body(x_ref, w_ref, b_ref, o_ref):
    i = pl.program_id(0)
    y = jnp.dot(x_ref[...], w_ref[...], preferred_element_type=_f32) + b_ref[...]
    y = jnp.maximum(y, 0.0)
    rows = i * NB + lax.broadcasted_iota(jnp.int32, (NB, HID), 0)
    o_ref[...] = jnp.where(rows < N, y, 0.0)


def _in_proj(x, w, b):
    return pl.pallas_call(
        _in_proj_body,
        grid=(NPAD // NB,),
        in_specs=[pl.BlockSpec((NB, HID), lambda i: (i, 0)),
                  pl.BlockSpec((HID, HID), lambda i: (0, 0)),
                  pl.BlockSpec((1, HID), lambda i: (0, 0))],
        out_specs=pl.BlockSpec((NB, HID), lambda i: (i, 0)),
        out_shape=jax.ShapeDtypeStruct((NPAD, HID), _f32),
    )(x, w, b)


def _qkv_body(x_ref, w_ref, q_ref, k_ref, v_ref):
    y = jnp.dot(x_ref[...], w_ref[...], preferred_element_type=_f32)
    q_ref[...] = y[:, 0:HID]
    k_ref[...] = y[:, HID:2 * HID]
    v_ref[...] = y[:, 2 * HID:3 * HID]


def _qkv(x, wcat):
    return pl.pallas_call(
        _qkv_body,
        grid=(NPAD // NB,),
        in_specs=[pl.BlockSpec((NB, HID), lambda i: (i, 0)),
                  pl.BlockSpec((HID, 3 * HID), lambda i: (0, 0))],
        out_specs=[pl.BlockSpec((NB, HID), lambda i: (i, 0)),
                   pl.BlockSpec((NB, HID), lambda i: (i, 0)),
                   pl.BlockSpec((NB, HID), lambda i: (i, 0))],
        out_shape=[jax.ShapeDtypeStruct((NPAD, HID), _f32),
                   jax.ShapeDtypeStruct((NPAD, HID), _f32),
                   jax.ShapeDtypeStruct((NPAD, HID), _f32)],
    )(x, wcat)


def _agg_body(n_ref, s_ref, x_ref, w_ref, a_ref, o_ref):
    i = pl.program_id(0)
    num = jnp.concatenate([n_ref[h] for h in range(HEADS)], axis=1)
    den = jnp.concatenate(
        [jnp.broadcast_to(s_ref[h][:, None], (NB, D)) for h in range(HEADS)], axis=1)
    agg = num / (den + 1e-16)
    g = jax.nn.gelu(agg)
    out = jnp.dot(g, w_ref[...], preferred_element_type=_f32)
    a = a_ref[0, 0]
    y = a * out + (1.0 - a) * x_ref[...]
    rows = i * NB + lax.broadcasted_iota(jnp.int32, (NB, HID), 0)
    o_ref[...] = jnp.where(rows < N, y, 0.0)


def _agg_update(numer, ssum, x, wa, a_gate):
    return pl.pallas_call(
        _agg_body,
        grid=(NPAD // NB,),
        in_specs=[pl.BlockSpec((HEADS, NB, D), lambda i: (0, i, 0)),
                  pl.BlockSpec((HEADS, NB), lambda i: (0, i)),
                  pl.BlockSpec((NB, HID), lambda i: (i, 0)),
                  pl.BlockSpec((HID, HID), lambda i: (0, 0)),
                  pl.BlockSpec((1, 1), lambda i: (0, 0))],
        out_specs=pl.BlockSpec((NB, HID), lambda i: (i, 0)),
        out_shape=jax.ShapeDtypeStruct((NPAD, HID), _f32),
    )(numer, ssum, x, wa, a_gate)


def _out_proj_body(x_ref, w_ref, b_ref, o_ref):
    y = jnp.dot(x_ref[...], w_ref[...], preferred_element_type=_f32) + b_ref[...]
    o_ref[...] = jnp.maximum(y, 0.0)


def _out_proj(x, w, b):
    return pl.pallas_call(
        _out_proj_body,
        grid=(NPAD // NB,),
        in_specs=[pl.BlockSpec((NB, HID), lambda i: (i, 0)),
                  pl.BlockSpec((HID, HID), lambda i: (0, 0)),
                  pl.BlockSpec((1, HID), lambda i: (0, 0))],
        out_specs=pl.BlockSpec((NB, HID), lambda i: (i, 0)),
        out_shape=jax.ShapeDtypeStruct((N, HID), _f32),
    )(x, w, b)


@functools.partial(
    pl.kernel,
    out_type=jax.ShapeDtypeStruct((HEADS * EP,), _f32),
    mesh=_scmesh,
    scratch_types=[[pltpu.VMEM((CA,), jnp.int32) for _ in range(2)],
                   [pltpu.VMEM((CA,), jnp.int32) for _ in range(2)],
                   [pltpu.VMEM((CA, HID), _f32) for _ in range(2)],
                   [pltpu.VMEM((CA, HID), _f32) for _ in range(2)],
                   [pltpu.VMEM((EPT,), _f32) for _ in range(HEADS)],
                   [pltpu.SemaphoreType.DMA for _ in range(2)],
                   [pltpu.SemaphoreType.DMA for _ in range(2)]],
    compiler_params=pltpu.CompilerParams(needs_layout_passes=False),
)
def _pass_a(q_hbm, k_hbm, src_hbm, dst_hbm, alpha_out,
            srcv, dstv, krows, qrows, astage, semi, semg):
    wid = lax.axis_index("s") * 2 + lax.axis_index("c")
    ebase = wid * EPT
    iota = lax.iota(jnp.int32, 16)
    NCHUNK = EPT // CA

    pltpu.sync_copy(src_hbm.at[pl.ds(ebase, CA)], srcv[0])
    pltpu.sync_copy(dst_hbm.at[pl.ds(ebase, CA)], dstv[0])
    pltpu.async_copy(k_hbm.at[srcv[0]], krows[0], semg[0])
    pltpu.async_copy(q_hbm.at[dstv[0]], qrows[0], semg[0])

    def do_chunk(g, b):
        nxt = 1 - b
        gn = jnp.minimum(g + 1, NCHUNK - 1)
        cbn = ebase + gn * CA
        pltpu.async_copy(src_hbm.at[pl.ds(cbn, CA)], srcv[nxt], semi[nxt])
        pltpu.async_copy(dst_hbm.at[pl.ds(cbn, CA)], dstv[nxt], semi[nxt])
        pltpu.make_async_copy(k_hbm.at[srcv[b]], krows[b], semg[b]).wait()
        pltpu.make_async_copy(q_hbm.at[dstv[b]], qrows[b], semg[b]).wait()

        def group_body(t, _):
            rows = iota + t * 16
            off = g * CA + t * 16
            for h in range(HEADS):
                acc = None
                for d in range(D):
                    cols = jnp.full((16,), h * D + d, jnp.int32)
                    qv = plsc.load_gather(qrows[b], [rows, cols])
                    kv = plsc.load_gather(krows[b], [rows, cols])
                    p = qv * kv
                    acc = p if acc is None else acc + p
                astage[h][pl.ds(off, 16)] = acc
            return 0

        lax.fori_loop(0, CA // 16, group_body, 0)
        pltpu.make_async_copy(src_hbm.at[pl.ds(0, CA)], srcv[nxt], semi[nxt]).wait()
        pltpu.make_async_copy(dst_hbm.at[pl.ds(0, CA)], dstv[nxt], semi[nxt]).wait()
        pltpu.async_copy(k_hbm.at[srcv[nxt]], krows[nxt], semg[nxt])
        pltpu.async_copy(q_hbm.at[dstv[nxt]], qrows[nxt], semg[nxt])

    def pair_body(j, _):
        do_chunk(2 * j, 0)
        do_chunk(2 * j + 1, 1)
        return 0

    lax.fori_loop(0, NCHUNK // 2, pair_body, 0)
    pltpu.make_async_copy(k_hbm.at[srcv[0]], krows[0], semg[0]).wait()
    pltpu.make_async_copy(q_hbm.at[dstv[0]], qrows[0], semg[0]).wait()
    for h in range(HEADS):
        pltpu.sync_copy(astage[h], alpha_out.at[pl.ds(h * EP + ebase, EPT)])


QN = NPAD // 4
JR = 256


@functools.partial(
    pl.kernel,
    out_type=[jax.ShapeDtypeStruct((HEADS * NPAD, D), _f32),
              jax.ShapeDtypeStruct((HEADS * NPAD,), _f32)],
    mesh=_scmesh,
    scratch_types=[[pltpu.VMEM((CB,), jnp.int32) for _ in range(4)],
                   [pltpu.VMEM((CB,), jnp.int32) for _ in range(4)],
                   [pltpu.VMEM((CB,), _f32) for _ in range(4)],
                   [pltpu.VMEM((CB, HID), _f32) for _ in range(2)],
                   [pltpu.VMEM((CB, D), _f32) for _ in range(2)],
                   pltpu.VMEM((160, D), _f32),
                   pltpu.VMEM((160,), _f32),
                   pltpu.VMEM((160,), jnp.int32),
                   pltpu.VMEM_SHARED((QN + JR, D), _f32),
                   pltpu.VMEM_SHARED((QN + JR,), _f32),
                   [pltpu.SemaphoreType.DMA for _ in range(4)],
                   [pltpu.SemaphoreType.DMA for _ in range(2)],
                   [pltpu.SemaphoreType.DMA for _ in range(2)],
                   pltpu.SemaphoreType.DMA],
    compiler_params=pltpu.CompilerParams(needs_layout_passes=False),
)
def _pass_b(v_hbm, src_hbm, dst_hbm, alpha_hbm, z2_hbm, z1_hbm,
            numer_out, ssum_out, srcv, dstv, ev, vrows, msgbuf, stg, stg1,
            idbuf, acc, accs, semi, semg, sema, sem):
    c = lax.axis_index("c")
    s = lax.axis_index("s")
    iota = lax.iota(jnp.int32, 16)
    zrpt = (QN + JR) // 16
    NCHUNK = EPB // CB

    def issue_idx(g, b):
        base = s * EPB + g * CB
        cp1 = pltpu.async_copy(src_hbm.at[pl.ds(base, CB)], srcv[b], semi[b])
        cp2 = pltpu.async_copy(dst_hbm.at[pl.ds(base, CB)], dstv[b], semi[b])
        return cp1, cp2

    def phase_body(p, _):
        h = c * 2 + p // 4
        lo = (p % 4) * QN
        pltpu.sync_copy(z2_hbm.at[pl.ds(0, 160)], stg)
        pltpu.sync_copy(z1_hbm.at[pl.ds(0, 160)], stg1)
        for zb in range(5):
            zbase = s * zrpt + zb * 160

            def idfill(i, _):
                idbuf[pl.ds(i * 16, 16)] = iota + (zbase + i * 16)
                return 0

            lax.fori_loop(0, 10, idfill, 0)
            pltpu.sync_copy(stg, acc.at[idbuf])
            pltpu.sync_copy(stg1, accs.at[idbuf])
        plsc.subcore_barrier()

        def stage_idx(g, ib):
            base = s * EPB + g * CB
            pltpu.async_copy(src_hbm.at[pl.ds(base, CB)], srcv[ib], semi[ib])
            pltpu.async_copy(dst_hbm.at[pl.ds(base, CB)], dstv[ib], semi[ib])
            pltpu.async_copy(alpha_hbm.at[pl.ds(h * EP + base, CB)], ev[ib], semi[ib])

        def wait_idx(ib):
            pltpu.make_async_copy(src_hbm.at[pl.ds(0, CB)], srcv[ib], semi[ib]).wait()
            pltpu.make_async_copy(dst_hbm.at[pl.ds(0, CB)], dstv[ib], semi[ib]).wait()
            pltpu.make_async_copy(alpha_hbm.at[pl.ds(0, CB)], ev[ib], semi[ib]).wait()

        def do_chunk(g, ib, vb, first=False):
            if not first:
                pltpu.make_async_copy(msgbuf[vb], acc.at[dstv[ib]], sema[vb]).wait()
                pltpu.make_async_copy(ev[ib], accs.at[dstv[ib]], sema[vb]).wait()
            gn = jnp.minimum(g + 1, NCHUNK - 1)
            ibn = (ib + 1) % 4
            wait_idx(ibn)
            pltpu.async_copy(v_hbm.at[srcv[ibn]], vrows[1 - vb], semg[1 - vb])
            stage_idx(jnp.minimum(g + 2, NCHUNK - 1), (ib + 2) % 4)
            pltpu.make_async_copy(v_hbm.at[srcv[ib]], vrows[vb], semg[vb]).wait()

            def group_body(t, _):
                rows = iota + t * 16
                dl = dstv[ib][pl.ds(t * 16, 16)] - lo
                msk = (dl >= 0) & (dl < QN)
                junk = QN + ((iota + t * 16) & (JR - 1))
                dstv[ib][pl.ds(t * 16, 16)] = jnp.where(msk, dl, junk)
                e16 = jnp.exp(ev[ib][pl.ds(t * 16, 16)])
                ev[ib][pl.ds(t * 16, 16)] = e16
                for d in range(D):
                    colv = jnp.full((16,), d, jnp.int32) + h * D
                    cold = jnp.full((16,), d, jnp.int32)
                    vv = plsc.load_gather(vrows[vb], [rows, colv])
                    plsc.store_scatter(msgbuf[vb], [rows, cold], vv * e16)
                return 0

            lax.fori_loop(0, CB // 16, group_body, 0)
            pltpu.async_copy(msgbuf[vb], acc.at[dstv[ib]], sema[vb], add=True)
            pltpu.async_copy(ev[ib], accs.at[dstv[ib]], sema[vb], add=True)

        stage_idx(0, 0)
        stage_idx(1, 1)
        wait_idx(0)
        pltpu.async_copy(v_hbm.at[srcv[0]], vrows[0], semg[0])

        do_chunk(0, 0, 0, first=True)
        do_chunk(1, 1, 1, first=True)
        do_chunk(2, 2, 0)
        do_chunk(3, 3, 1)

        def quad_body(j, _):
            do_chunk(4 * j, 0, 0)
            do_chunk(4 * j + 1, 1, 1)
            do_chunk(4 * j + 2, 2, 0)
            do_chunk(4 * j + 3, 3, 1)
            return 0

        lax.fori_loop(1, NCHUNK // 4, quad_body, 0)
        pltpu.make_async_copy(msgbuf[0], acc.at[dstv[2]], sema[0]).wait()
        pltpu.make_async_copy(ev[2], accs.at[dstv[2]], sema[0]).wait()
        pltpu.make_async_copy(msgbuf[1], acc.at[dstv[3]], sema[1]).wait()
        pltpu.make_async_copy(ev[3], accs.at[dstv[3]], sema[1]).wait()
        pltpu.make_async_copy(v_hbm.at[srcv[0]], vrows[0], semg[0]).wait()
        wait_idx(1)
        plsc.subcore_barrier()
        for wb, wn in ((0, 160), (160, 160), (320, 160), (480, 160), (640, 144)):
            wbase = s * (QN // 16) + wb
            slo = pl.ds(h * NPAD + lo + wbase, wn)

            def idfill2(i, _):
                idbuf[pl.ds(i * 16, 16)] = iota + (wbase + i * 16)
                return 0

            lax.fori_loop(0, wn // 16, idfill2, 0)
            pltpu.async_copy(acc.at[idbuf.at[pl.ds(0, wn)]], stg.at[pl.ds(0, wn)], sem).wait()
            pltpu.sync_copy(stg.at[pl.ds(0, wn)], numer_out.at[slo])
            pltpu.async_copy(accs.at[idbuf.at[pl.ds(0, wn)]], stg1.at[pl.ds(0, wn)], sem).wait()
            pltpu.sync_copy(stg1.at[pl.ds(0, wn)], ssum_out.at[slo])
        plsc.subcore_barrier()
        return 0

    lax.fori_loop(0, 8, phase_body, 0)


def _fold_kv(w, rel, prel=None):
    wh = w.reshape(HID, HEADS, D)
    out = jnp.einsum("ihd,hde->ihe", wh, rel)
    if prel is not None:
        out = out * (prel[None, :, None] / np.sqrt(D))
    return out.reshape(HID, HID)


def _pad_edges(ei):
    src, dst = ei[0], ei[1]
    npad = EP - E
    i = jnp.arange(npad)
    src_p = jnp.concatenate([src, (i % N).astype(src.dtype)])
    dst_p = jnp.concatenate([dst, (N + i % (NPAD - N)).astype(dst.dtype)])
    return src_p.astype(jnp.int32), dst_p.astype(jnp.int32)


def kernel(x_author, x_paper, ei_writes, ei_rev_writes, Win_author, bin_author, Wout_author, bout_author, Win_paper, bin_paper, Wout_paper, bout_paper, Wk0_author, Wq0_author, Wv0_author, Wa0_author, skip0_author, Wk0_paper, Wq0_paper, Wv0_paper, Wa0_paper, skip0_paper, arel0_writes, mrel0_writes, prel0_writes, arel0_rev_writes, mrel0_rev_writes, prel0_rev_writes, Wk1_author, Wq1_author, Wv1_author, Wa1_author, skip1_author, Wk1_paper, Wq1_paper, Wv1_paper, Wa1_paper, skip1_paper, arel1_writes, mrel1_writes, prel1_writes, arel1_rev_writes, mrel1_rev_writes, prel1_rev_writes):
    p = dict(locals())
    types = ["author", "paper"]
    src_rel = {"author": "writes", "paper": "rev_writes"}
    edges = {"writes": _pad_edges(ei_writes), "rev_writes": _pad_edges(ei_rev_writes)}
    z2 = jax.lax.optimization_barrier(jnp.zeros((NPAD, D), _f32))
    z1 = jax.lax.optimization_barrier(jnp.zeros((NPAD,), _f32))

    x = {}
    for t in types:
        xp = jnp.pad(p["x_" + t], ((0, NPAD - N), (0, 0)))
        x[t] = _in_proj(xp, p["Win_" + t], p["bin_" + t].reshape(1, HID))

    for l in range(2):
        q, k, v = {}, {}, {}
        for t in types:
            r = src_rel[t]
            wk = _fold_kv(p["Wk%d_%s" % (l, t)], p["arel%d_%s" % (l, r)],
                          p["prel%d_%s" % (l, r)])
            wv = _fold_kv(p["Wv%d_%s" % (l, t)], p["mrel%d_%s" % (l, r)])
            wcat = jnp.concatenate([p["Wq%d_%s" % (l, t)], wk, wv], axis=1)
            q[t], k[t], v[t] = _qkv(x[t], wcat)

        agg = {}
        for (st, r, dt) in [("author", "writes", "paper"),
                            ("paper", "rev_writes", "author")]:
            src_p, dst_p = edges[r]
            alpha = _pass_a(q[dt], k[st], src_p, dst_p)
            numer, ssum = _pass_b(v[st], src_p, dst_p, alpha, z2, z1)
            agg[dt] = (numer.reshape(HEADS, NPAD, D), ssum.reshape(HEADS, NPAD))

        new_x = {}
        for t in types:
            a_gate = jax.nn.sigmoid(p["skip%d_%s" % (l, t)]).reshape(1, 1)
            numer, ssum = agg[t]
            new_x[t] = _agg_update(numer, ssum, x[t], p["Wa%d_%s" % (l, t)], a_gate)
        x = new_x

    outs = []
    for t in types:
        outs.append(_out_proj(x[t], p["Wout_" + t], p["bout_" + t].reshape(1, HID)))
    return (outs[0], outs[1])
